# baseline (speedup 1.0000x reference)
import functools

import jax
import jax.numpy as jnp
from jax import lax
from jax.experimental import pallas as pl
from jax.experimental.pallas import tpu as pltpu
from jax.experimental.pallas import tpu_sc as plsc

N_NODES = 10000
N_EDGES = 320000
DIM = 128
N_GRAPHS = 64
N_OUT = 16

NUM_CORES = 2
NUM_SUBCORES = 16
NUM_TILES = NUM_CORES * NUM_SUBCORES
EDGES_PER_TILE = N_EDGES // NUM_TILES
CHUNK = 80
N_CHUNKS = EDGES_PER_TILE // CHUNK
N_PHASES = 5
CHUNKS_PER_PHASE = N_CHUNKS // N_PHASES
ROWS_PER_SUBCORE = 624
TAIL_ROW0 = ROWS_PER_SUBCORE * NUM_SUBCORES
TAIL_ROWS = N_NODES - TAIL_ROW0


def _sc_agg_body(h_hbm, src_hbm, dst_hbm, zeros_hbm, out_hbm,
                 src_a, dst_a,
                 rows0, rows1, rows2, rows3, agg_sh,
                 sem_i, sem_g0, sem_g1, sem_g2, sem_g3,
                 sem_s0, sem_s1, sem_s2, sem_s3):
    c = lax.axis_index("c")
    s = lax.axis_index("s")
    wid = c * NUM_SUBCORES + s
    r0 = s * ROWS_PER_SUBCORE

    idx_bufs = ((src_a, dst_a), (src_a, dst_a))

    def stage_idx(ph, pair):
        pltpu.async_copy(src_hbm.at[wid, ph], pair[0], sem_i)
        pltpu.async_copy(dst_hbm.at[wid, ph], pair[1], sem_i)

    def wait_idx(pair):
        pltpu.make_async_copy(src_hbm.at[wid, 0], pair[0], sem_i).wait()
        pltpu.make_async_copy(dst_hbm.at[wid, 0], pair[1], sem_i).wait()

    stage_idx(0, idx_bufs[0])

    pltpu.sync_copy(zeros_hbm.at[pl.ds(r0, ROWS_PER_SUBCORE)],
                    agg_sh.at[pl.ds(r0, ROWS_PER_SUBCORE)])

    @pl.when(s == 0)
    def _():
        pltpu.sync_copy(zeros_hbm.at[pl.ds(TAIL_ROW0, TAIL_ROWS)],
                        agg_sh.at[pl.ds(TAIL_ROW0, TAIL_ROWS)])

    plsc.subcore_barrier()

    rows = (rows0, rows1, rows2, rows3)
    sem_g = (sem_g0, sem_g1, sem_g2, sem_g3)
    sem_s = (sem_s0, sem_s1, sem_s2, sem_s3)

    def gather(src_p, j, b):
        pltpu.async_copy(h_hbm.at[src_p.at[j]], rows[b], sem_g[b])

    def gather_wait(src_p, b):
        pltpu.make_async_copy(h_hbm.at[src_p.at[0]], rows[b],
                              sem_g[b]).wait()

    def scatter(dst_p, j, b):
        pltpu.async_copy(rows[b], agg_sh.at[dst_p.at[j]], sem_s[b],
                         add=True)

    def scatter_wait(dst_p, b):
        pltpu.make_async_copy(rows[b], agg_sh.at[dst_p.at[0]],
                              sem_s[b]).wait()

    for ph in range(N_PHASES):
        src_p, dst_p = idx_bufs[ph % 2]
        wait_idx(idx_bufs[ph % 2])

        gather(src_p, 0, 0)
        gather(src_p, 1, 1)

        def group_body(kk, carry, src_p=src_p, dst_p=dst_p):
            j0 = 4 * kk
            for b in range(4):
                j = j0 + b
                gather_wait(src_p, b)
                scatter(dst_p, j, b)
                b2 = (b + 2) % 4

                @pl.when(j + 2 < CHUNKS_PER_PHASE)
                def _(j=j, b2=b2):
                    @pl.when(j >= 2)
                    def _():
                        scatter_wait(dst_p, b2)

                    gather(src_p, j + 2, b2)

            return carry

        lax.fori_loop(0, CHUNKS_PER_PHASE // 4, group_body, 0)
        gather_wait(src_p, 0)
        scatter(dst_p, CHUNKS_PER_PHASE - 1, 0)
        for b in range(4):
            scatter_wait(dst_p, b)
        if ph + 1 < N_PHASES:
            stage_idx(ph + 1, idx_bufs[(ph + 1) % 2])

    plsc.subcore_barrier()

    pltpu.sync_copy(agg_sh.at[pl.ds(r0, ROWS_PER_SUBCORE)],
                    out_hbm.at[c, pl.ds(r0, ROWS_PER_SUBCORE)])

    @pl.when(s == 0)
    def _():
        pltpu.sync_copy(agg_sh.at[pl.ds(TAIL_ROW0, TAIL_ROWS)],
                        out_hbm.at[c, pl.ds(TAIL_ROW0, TAIL_ROWS)])


@functools.cache
def _get_sc_agg():
    return pl.kernel(
        _sc_agg_body,
        out_type=jax.ShapeDtypeStruct((NUM_CORES, N_NODES, DIM), jnp.float32),
        mesh=plsc.VectorSubcoreMesh(core_axis_name="c", subcore_axis_name="s",
                                    num_cores=NUM_CORES,
                                    num_subcores=NUM_SUBCORES),
        scratch_types=(
            [pltpu.VMEM((CHUNKS_PER_PHASE, CHUNK), jnp.int32)] * 2
            + [pltpu.VMEM((CHUNK, DIM), jnp.float32)] * 4
            + [pltpu.VMEM_SHARED((N_NODES, DIM), jnp.float32)]
            + [pltpu.SemaphoreType.DMA] * 9
        ),
    )


def _bn(z, g, b):
    m = jnp.mean(z, axis=0, keepdims=True)
    v = jnp.mean((z - m) * (z - m), axis=0, keepdims=True)
    return (z - m) * lax.rsqrt(v + 1e-5) * g + b


def _tc_layer_body(h_ref, agg_ref, w1_ref, b1_ref, g1_ref, be1_ref,
                   w2_ref, b2_ref, g2_ref, be2_ref, out_ref):
    a = agg_ref[...]
    z = h_ref[...] + a[0] + a[1]
    z = jnp.dot(z, w1_ref[...], preferred_element_type=jnp.float32) + b1_ref[...]
    z = jnp.maximum(_bn(z, g1_ref[...], be1_ref[...]), 0.0)
    z = jnp.dot(z, w2_ref[...], preferred_element_type=jnp.float32) + b2_ref[...]
    z = jnp.maximum(_bn(z, g2_ref[...], be2_ref[...]), 0.0)
    out_ref[...] = z


def _tc_final_body(h_ref, agg_ref, w1_ref, b1_ref, g1_ref, be1_ref,
                   w2_ref, b2_ref, g2_ref, be2_ref,
                   batch_ref, clsw_ref, clsb_ref, out_ref):
    a = agg_ref[...]
    z = h_ref[...] + a[0] + a[1]
    z = jnp.dot(z, w1_ref[...], preferred_element_type=jnp.float32) + b1_ref[...]
    z = jnp.maximum(_bn(z, g1_ref[...], be1_ref[...]), 0.0)
    z = jnp.dot(z, w2_ref[...], preferred_element_type=jnp.float32) + b2_ref[...]
    z = _bn(z, g2_ref[...], be2_ref[...])

    ids = lax.broadcasted_iota(jnp.int32, (N_NODES, N_GRAPHS), 1)
    onehot = (batch_ref[...] == ids).astype(jnp.float32)
    dnums = (((0,), (0,)), ((), ()))
    sums = lax.dot_general(onehot, z, dnums,
                           preferred_element_type=jnp.float32)
    cnts = lax.dot_general(onehot, jnp.ones((N_NODES, 1), jnp.float32),
                           dnums, preferred_element_type=jnp.float32)
    readout = sums / jnp.maximum(cnts, 1.0)
    out_ref[...] = (jnp.dot(readout, clsw_ref[...],
                            preferred_element_type=jnp.float32)
                    + clsb_ref[...])


_tc_layer = pl.pallas_call(
    _tc_layer_body,
    out_shape=jax.ShapeDtypeStruct((N_NODES, DIM), jnp.float32),
)

_tc_final = pl.pallas_call(
    _tc_final_body,
    out_shape=jax.ShapeDtypeStruct((N_GRAPHS, N_OUT), jnp.float32),
)


def kernel(x, edge_index, batch, params):
    src = edge_index[0]
    dst = edge_index[1]
    src3 = src.reshape(NUM_TILES, N_PHASES, CHUNKS_PER_PHASE, CHUNK)
    dst3 = dst.reshape(NUM_TILES, N_PHASES, CHUNKS_PER_PHASE, CHUNK)
    zeros = jnp.zeros((N_NODES, DIM), jnp.float32)
    batch2d = batch.reshape(N_NODES, 1).astype(jnp.int32)

    h = x
    layers = params["layers"]
    out = None
    for i, p in enumerate(layers):
        aggs = _get_sc_agg()(h, src3, dst3, zeros)
        w = (p["W1"], p["b1"].reshape(1, -1), p["g1"].reshape(1, -1),
             p["be1"].reshape(1, -1), p["W2"], p["b2"].reshape(1, -1),
             p["g2"].reshape(1, -1), p["be2"].reshape(1, -1))
        if i != len(layers) - 1:
            h = _tc_layer(h, aggs, *w)
        else:
            out = _tc_final(h, aggs, *w, batch2d, params["cls_W"],
                            params["cls_b"].reshape(1, -1))
    return out


# speedup vs baseline: 1.1221x; 1.1221x over previous
import functools

import jax
import jax.numpy as jnp
from jax import lax
from jax.experimental import pallas as pl
from jax.experimental.pallas import tpu as pltpu
from jax.experimental.pallas import tpu_sc as plsc

N_NODES = 10000
N_EDGES = 320000
DIM = 128
N_GRAPHS = 64
N_OUT = 16

NUM_CORES = 2
NUM_SUBCORES = 16
NUM_TILES = NUM_CORES * NUM_SUBCORES
EDGES_PER_TILE = N_EDGES // NUM_TILES
CHUNK = 80
N_CHUNKS = EDGES_PER_TILE // CHUNK
N_PHASES = 5
CHUNKS_PER_PHASE = N_CHUNKS // N_PHASES
ROWS_PER_SUBCORE = 624
TAIL_ROW0 = ROWS_PER_SUBCORE * NUM_SUBCORES
TAIL_ROWS = N_NODES - TAIL_ROW0


def _sc_agg_body(h_hbm, src_hbm, dst_hbm, zeros_hbm, out_hbm,
                 src_a, dst_a,
                 rows0, rows1, rows2, rows3, agg_sh,
                 sem_i, sem_g0, sem_g1, sem_g2, sem_g3):
    c = lax.axis_index("c")
    s = lax.axis_index("s")
    wid = c * NUM_SUBCORES + s
    r0 = s * ROWS_PER_SUBCORE

    idx_bufs = ((src_a, dst_a), (src_a, dst_a))

    def stage_idx(ph, pair):
        pltpu.async_copy(src_hbm.at[wid, ph], pair[0], sem_i)
        pltpu.async_copy(dst_hbm.at[wid, ph], pair[1], sem_i)

    def wait_idx(pair):
        pltpu.make_async_copy(src_hbm.at[wid, 0], pair[0], sem_i).wait()
        pltpu.make_async_copy(dst_hbm.at[wid, 0], pair[1], sem_i).wait()

    stage_idx(0, idx_bufs[0])

    pltpu.sync_copy(zeros_hbm.at[pl.ds(r0, ROWS_PER_SUBCORE)],
                    agg_sh.at[pl.ds(r0, ROWS_PER_SUBCORE)])

    @pl.when(s == 0)
    def _():
        pltpu.sync_copy(zeros_hbm.at[pl.ds(TAIL_ROW0, TAIL_ROWS)],
                        agg_sh.at[pl.ds(TAIL_ROW0, TAIL_ROWS)])

    plsc.subcore_barrier()

    rows = (rows0, rows1, rows2, rows3)
    sem_g = (sem_g0, sem_g1, sem_g2, sem_g3)

    def gather(src_p, j, b):
        pltpu.async_copy(h_hbm.at[src_p.at[j]], rows[b], sem_g[b])

    def gather_wait(src_p, b):
        pltpu.make_async_copy(h_hbm.at[src_p.at[0]], rows[b],
                              sem_g[b]).wait()

    def scatter(dst_p, j, b):
        pltpu.sync_copy(rows[b], agg_sh.at[dst_p.at[j]], add=True)

    for ph in range(N_PHASES):
        src_p, dst_p = idx_bufs[ph % 2]
        wait_idx(idx_bufs[ph % 2])

        for b in range(4):
            gather(src_p, b, b)

        def group_body(kk, carry, src_p=src_p, dst_p=dst_p):
            j0 = 4 * kk
            for b in range(4):
                gather_wait(src_p, b)
                scatter(dst_p, j0 + b, b)

                @pl.when(j0 + b + 4 < CHUNKS_PER_PHASE)
                def _(b=b):
                    gather(src_p, j0 + b + 4, b)

            return carry

        lax.fori_loop(0, CHUNKS_PER_PHASE // 4, group_body, 0)
        gather_wait(src_p, 0)
        scatter(dst_p, CHUNKS_PER_PHASE - 1, 0)
        if ph + 1 < N_PHASES:
            stage_idx(ph + 1, idx_bufs[(ph + 1) % 2])

    plsc.subcore_barrier()

    pltpu.sync_copy(agg_sh.at[pl.ds(r0, ROWS_PER_SUBCORE)],
                    out_hbm.at[c, pl.ds(r0, ROWS_PER_SUBCORE)])

    @pl.when(s == 0)
    def _():
        pltpu.sync_copy(agg_sh.at[pl.ds(TAIL_ROW0, TAIL_ROWS)],
                        out_hbm.at[c, pl.ds(TAIL_ROW0, TAIL_ROWS)])


@functools.cache
def _get_sc_agg():
    return pl.kernel(
        _sc_agg_body,
        out_type=jax.ShapeDtypeStruct((NUM_CORES, N_NODES, DIM), jnp.float32),
        mesh=plsc.VectorSubcoreMesh(core_axis_name="c", subcore_axis_name="s",
                                    num_cores=NUM_CORES,
                                    num_subcores=NUM_SUBCORES),
        scratch_types=(
            [pltpu.VMEM((CHUNKS_PER_PHASE, CHUNK), jnp.int32)] * 2
            + [pltpu.VMEM((CHUNK, DIM), jnp.float32)] * 4
            + [pltpu.VMEM_SHARED((N_NODES, DIM), jnp.float32)]
            + [pltpu.SemaphoreType.DMA] * 5
        ),
    )


def _bn(z, g, b):
    m = jnp.mean(z, axis=0, keepdims=True)
    v = jnp.mean((z - m) * (z - m), axis=0, keepdims=True)
    return (z - m) * lax.rsqrt(v + 1e-5) * g + b


def _tc_layer_body(h_ref, agg_ref, w1_ref, b1_ref, g1_ref, be1_ref,
                   w2_ref, b2_ref, g2_ref, be2_ref, out_ref):
    a = agg_ref[...]
    z = h_ref[...] + a[0] + a[1]
    z = jnp.dot(z, w1_ref[...], preferred_element_type=jnp.float32) + b1_ref[...]
    z = jnp.maximum(_bn(z, g1_ref[...], be1_ref[...]), 0.0)
    z = jnp.dot(z, w2_ref[...], preferred_element_type=jnp.float32) + b2_ref[...]
    z = jnp.maximum(_bn(z, g2_ref[...], be2_ref[...]), 0.0)
    out_ref[...] = z


def _tc_final_body(h_ref, agg_ref, w1_ref, b1_ref, g1_ref, be1_ref,
                   w2_ref, b2_ref, g2_ref, be2_ref,
                   batch_ref, clsw_ref, clsb_ref, out_ref):
    a = agg_ref[...]
    z = h_ref[...] + a[0] + a[1]
    z = jnp.dot(z, w1_ref[...], preferred_element_type=jnp.float32) + b1_ref[...]
    z = jnp.maximum(_bn(z, g1_ref[...], be1_ref[...]), 0.0)
    z = jnp.dot(z, w2_ref[...], preferred_element_type=jnp.float32) + b2_ref[...]
    z = _bn(z, g2_ref[...], be2_ref[...])

    ids = lax.broadcasted_iota(jnp.int32, (N_NODES, N_GRAPHS), 1)
    onehot = (batch_ref[...] == ids).astype(jnp.float32)
    dnums = (((0,), (0,)), ((), ()))
    sums = lax.dot_general(onehot, z, dnums,
                           preferred_element_type=jnp.float32)
    cnts = lax.dot_general(onehot, jnp.ones((N_NODES, 1), jnp.float32),
                           dnums, preferred_element_type=jnp.float32)
    readout = sums / jnp.maximum(cnts, 1.0)
    out_ref[...] = (jnp.dot(readout, clsw_ref[...],
                            preferred_element_type=jnp.float32)
                    + clsb_ref[...])


_tc_layer = pl.pallas_call(
    _tc_layer_body,
    out_shape=jax.ShapeDtypeStruct((N_NODES, DIM), jnp.float32),
)

_tc_final = pl.pallas_call(
    _tc_final_body,
    out_shape=jax.ShapeDtypeStruct((N_GRAPHS, N_OUT), jnp.float32),
)


def kernel(x, edge_index, batch, params):
    src = edge_index[0]
    dst = edge_index[1]
    src3 = src.reshape(NUM_TILES, N_PHASES, CHUNKS_PER_PHASE, CHUNK)
    dst3 = dst.reshape(NUM_TILES, N_PHASES, CHUNKS_PER_PHASE, CHUNK)
    zeros = jnp.zeros((N_NODES, DIM), jnp.float32)
    batch2d = batch.reshape(N_NODES, 1).astype(jnp.int32)

    h = x
    layers = params["layers"]
    out = None
    for i, p in enumerate(layers):
        aggs = _get_sc_agg()(h, src3, dst3, zeros)
        w = (p["W1"], p["b1"].reshape(1, -1), p["g1"].reshape(1, -1),
             p["be1"].reshape(1, -1), p["W2"], p["b2"].reshape(1, -1),
             p["g2"].reshape(1, -1), p["be2"].reshape(1, -1))
        if i != len(layers) - 1:
            h = _tc_layer(h, aggs, *w)
        else:
            out = _tc_final(h, aggs, *w, batch2d, params["cls_W"],
                            params["cls_b"].reshape(1, -1))
    return out


# speedup vs baseline: 1.1508x; 1.0256x over previous
import functools

import jax
import jax.numpy as jnp
from jax import lax
from jax.experimental import pallas as pl
from jax.experimental.pallas import tpu as pltpu
from jax.experimental.pallas import tpu_sc as plsc

N_NODES = 10000
N_EDGES = 320000
DIM = 128
N_GRAPHS = 64
N_OUT = 16

NUM_CORES = 2
NUM_SUBCORES = 16
NUM_TILES = NUM_CORES * NUM_SUBCORES
EDGES_PER_TILE = N_EDGES // NUM_TILES
CHUNK = 80
N_CHUNKS = EDGES_PER_TILE // CHUNK
N_PHASES = 5
CHUNKS_PER_PHASE = N_CHUNKS // N_PHASES
ROWS_PER_SUBCORE = 624
TAIL_ROW0 = ROWS_PER_SUBCORE * NUM_SUBCORES
TAIL_ROWS = N_NODES - TAIL_ROW0


def _sc_agg_body(h_hbm, ei_hbm, zeros_hbm, out_hbm,
                 src_a, dst_a,
                 rows0, rows1, rows2, rows3, agg_sh,
                 sem_i, sem_g0, sem_g1, sem_g2, sem_g3):
    c = lax.axis_index("c")
    s = lax.axis_index("s")
    wid = c * NUM_SUBCORES + s
    r0 = s * ROWS_PER_SUBCORE

    idx_bufs = ((src_a, dst_a), (src_a, dst_a))

    def stage_idx(ph, pair):
        pltpu.async_copy(ei_hbm.at[0, wid, ph], pair[0], sem_i)
        pltpu.async_copy(ei_hbm.at[1, wid, ph], pair[1], sem_i)

    def wait_idx(pair):
        pltpu.make_async_copy(ei_hbm.at[0, wid, 0], pair[0], sem_i).wait()
        pltpu.make_async_copy(ei_hbm.at[1, wid, 0], pair[1], sem_i).wait()

    stage_idx(0, idx_bufs[0])

    pltpu.sync_copy(zeros_hbm.at[pl.ds(r0, ROWS_PER_SUBCORE)],
                    agg_sh.at[pl.ds(r0, ROWS_PER_SUBCORE)])

    @pl.when(s == 0)
    def _():
        pltpu.sync_copy(zeros_hbm.at[pl.ds(TAIL_ROW0, TAIL_ROWS)],
                        agg_sh.at[pl.ds(TAIL_ROW0, TAIL_ROWS)])

    plsc.subcore_barrier()

    rows = (rows0, rows1, rows2, rows3)
    sem_g = (sem_g0, sem_g1, sem_g2, sem_g3)

    def gather(src_p, j, b):
        pltpu.async_copy(h_hbm.at[src_p.at[j]], rows[b], sem_g[b])

    def gather_wait(src_p, b):
        pltpu.make_async_copy(h_hbm.at[src_p.at[0]], rows[b],
                              sem_g[b]).wait()

    def scatter(dst_p, j, b):
        pltpu.sync_copy(rows[b], agg_sh.at[dst_p.at[j]], add=True)

    for ph in range(N_PHASES):
        src_p, dst_p = idx_bufs[ph % 2]
        wait_idx(idx_bufs[ph % 2])

        for b in range(4):
            gather(src_p, b, b)

        def group_body(kk, carry, src_p=src_p, dst_p=dst_p):
            j0 = 4 * kk
            for b in range(4):
                gather_wait(src_p, b)
                scatter(dst_p, j0 + b, b)

                @pl.when(j0 + b + 4 < CHUNKS_PER_PHASE)
                def _(b=b):
                    gather(src_p, j0 + b + 4, b)

            return carry

        lax.fori_loop(0, CHUNKS_PER_PHASE // 4, group_body, 0)
        gather_wait(src_p, 0)
        scatter(dst_p, CHUNKS_PER_PHASE - 1, 0)
        if ph + 1 < N_PHASES:
            stage_idx(ph + 1, idx_bufs[(ph + 1) % 2])

    plsc.subcore_barrier()

    pltpu.sync_copy(agg_sh.at[pl.ds(r0, ROWS_PER_SUBCORE)],
                    out_hbm.at[c, pl.ds(r0, ROWS_PER_SUBCORE)])

    @pl.when(s == 0)
    def _():
        pltpu.sync_copy(agg_sh.at[pl.ds(TAIL_ROW0, TAIL_ROWS)],
                        out_hbm.at[c, pl.ds(TAIL_ROW0, TAIL_ROWS)])


@functools.cache
def _get_sc_agg():
    return pl.kernel(
        _sc_agg_body,
        out_type=jax.ShapeDtypeStruct((NUM_CORES, N_NODES, DIM), jnp.float32),
        mesh=plsc.VectorSubcoreMesh(core_axis_name="c", subcore_axis_name="s",
                                    num_cores=NUM_CORES,
                                    num_subcores=NUM_SUBCORES),
        scratch_types=(
            [pltpu.VMEM((CHUNKS_PER_PHASE, CHUNK), jnp.int32)] * 2
            + [pltpu.VMEM((CHUNK, DIM), jnp.float32)] * 4
            + [pltpu.VMEM_SHARED((N_NODES, DIM), jnp.float32)]
            + [pltpu.SemaphoreType.DMA] * 5
        ),
    )


def _bn(z, g, b):
    m = jnp.mean(z, axis=0, keepdims=True)
    v = jnp.mean((z - m) * (z - m), axis=0, keepdims=True)
    return (z - m) * lax.rsqrt(v + 1e-5) * g + b


def _tc_layer_body(h_ref, agg_ref, w1_ref, b1_ref, g1_ref, be1_ref,
                   w2_ref, b2_ref, g2_ref, be2_ref, out_ref):
    a = agg_ref[...]
    z = h_ref[...] + a[0] + a[1]
    z = jnp.dot(z, w1_ref[...], preferred_element_type=jnp.float32) + b1_ref[...]
    z = jnp.maximum(_bn(z, g1_ref[...], be1_ref[...]), 0.0)
    z = jnp.dot(z, w2_ref[...], preferred_element_type=jnp.float32) + b2_ref[...]
    z = jnp.maximum(_bn(z, g2_ref[...], be2_ref[...]), 0.0)
    out_ref[...] = z


def _tc_final_body(h_ref, agg_ref, w1_ref, b1_ref, g1_ref, be1_ref,
                   w2_ref, b2_ref, g2_ref, be2_ref,
                   batch_ref, clsw_ref, clsb_ref, out_ref):
    a = agg_ref[...]
    z = h_ref[...] + a[0] + a[1]
    z = jnp.dot(z, w1_ref[...], preferred_element_type=jnp.float32) + b1_ref[...]
    z = jnp.maximum(_bn(z, g1_ref[...], be1_ref[...]), 0.0)
    z = jnp.dot(z, w2_ref[...], preferred_element_type=jnp.float32) + b2_ref[...]
    z = _bn(z, g2_ref[...], be2_ref[...])

    ids = lax.broadcasted_iota(jnp.int32, (N_NODES, N_GRAPHS), 1)
    onehot = (batch_ref[...] == ids).astype(jnp.float32)
    dnums = (((0,), (0,)), ((), ()))
    sums = lax.dot_general(onehot, z, dnums,
                           preferred_element_type=jnp.float32)
    cnts = lax.dot_general(onehot, jnp.ones((N_NODES, 1), jnp.float32),
                           dnums, preferred_element_type=jnp.float32)
    readout = sums / jnp.maximum(cnts, 1.0)
    out_ref[...] = (jnp.dot(readout, clsw_ref[...],
                            preferred_element_type=jnp.float32)
                    + clsb_ref[...])


_tc_layer = pl.pallas_call(
    _tc_layer_body,
    out_shape=jax.ShapeDtypeStruct((N_NODES, DIM), jnp.float32),
)

_tc_final = pl.pallas_call(
    _tc_final_body,
    out_shape=jax.ShapeDtypeStruct((N_GRAPHS, N_OUT), jnp.float32),
)


def kernel(x, edge_index, batch, params):
    ei5 = edge_index.reshape(2, NUM_TILES, N_PHASES, CHUNKS_PER_PHASE, CHUNK)
    zeros = jnp.zeros((N_NODES, DIM), jnp.float32)
    batch2d = batch.reshape(N_NODES, 1)

    h = x
    layers = params["layers"]
    out = None
    for i, p in enumerate(layers):
        aggs = _get_sc_agg()(h, ei5, zeros)
        w = (p["W1"], p["b1"], p["g1"], p["be1"],
             p["W2"], p["b2"], p["g2"], p["be2"])
        if i != len(layers) - 1:
            h = _tc_layer(h, aggs, *w)
        else:
            out = _tc_final(h, aggs, *w, batch2d, params["cls_W"],
                            params["cls_b"])
    return out
